# baseline (speedup 1.0000x reference)
import jax
import jax.numpy as jnp
from jax import lax
from jax.experimental import pallas as pl
from jax.experimental.pallas import tpu as pltpu
from jax.experimental.pallas import tpu_sc as plsc

_B, _NM, _NES = 4, 2500, 500
_FEAT_M = 48
_FEAT_E = 16
_HID, _NOUT, _NPASS = 64, 4, 3
_E_M2M, _E_E2M = 80000, 5000

_N = _B * _NM
_NE = _B * _NES
_W = 2 * _HID

_NC, _NS, _L = 2, 16, 16
_NW = _NC * _NS
_CHUNK = 128
_N_PAD = 10240
_ROWS_PER_TILE = _N_PAD // _NS

_EQ = _NW * _CHUNK * 8
_E1 = _B * _E_M2M
_E1_PAD = _EQ * ((_E1 + _EQ - 1) // _EQ)
_E2 = _B * _E_E2M
_E2_PAD = _EQ * ((_E2 + _EQ - 1) // _EQ)

_HP = jax.lax.Precision.HIGHEST
_f32 = jnp.float32
_bf16 = jnp.bfloat16

_MESH = plsc.VectorSubcoreMesh(core_axis_name="c", subcore_axis_name="s")


def _drain(src, dst, sem):
    pltpu.make_async_copy(src, dst, sem).wait()


def _make_gather(e_pad):
    per_worker = e_pad // _NW
    n_chunks = per_worker // _CHUNK
    assert n_chunks % 2 == 0 and (n_chunks % 8 == 0 or _NW * n_chunks % 8 == 0)

    def body(a_hbm, b_hbm, src_hbm, dst_hbm, g1_hbm, g2_hbm,
             srcv, dstv, ra0, rb0, ra1, rb1,
             sga0, sgb0, sga1, sgb1, swb):
        wid = lax.axis_index("s") * _NC + lax.axis_index("c")
        base_c = wid * n_chunks
        base_e = wid * per_worker
        pltpu.sync_copy(src_hbm.at[pl.ds(base_c, n_chunks)], srcv)
        pltpu.sync_copy(dst_hbm.at[pl.ds(base_c, n_chunks)], dstv)
        pltpu.async_copy(a_hbm.at[srcv.at[0]], ra0, sga0)
        pltpu.async_copy(b_hbm.at[dstv.at[0]], rb0, sgb0)
        pltpu.async_copy(a_hbm.at[srcv.at[1]], ra1, sga1)
        pltpu.async_copy(b_hbm.at[dstv.at[1]], rb1, sgb1)

        @pl.loop(0, n_chunks, step=2)
        def _(c0):
            c1 = c0 + 1
            _drain(a_hbm.at[srcv.at[c0]], ra0, sga0)
            _drain(b_hbm.at[dstv.at[c0]], rb0, sgb0)
            w1 = pltpu.async_copy(
                ra0, g1_hbm.at[pl.ds(base_e + c0 * _CHUNK, _CHUNK)], swb)
            w2 = pltpu.async_copy(
                rb0, g2_hbm.at[pl.ds(base_e + c0 * _CHUNK, _CHUNK)], swb)
            _drain(a_hbm.at[srcv.at[c1]], ra1, sga1)
            _drain(b_hbm.at[dstv.at[c1]], rb1, sgb1)
            w3 = pltpu.async_copy(
                ra1, g1_hbm.at[pl.ds(base_e + c1 * _CHUNK, _CHUNK)], swb)
            w4 = pltpu.async_copy(
                rb1, g2_hbm.at[pl.ds(base_e + c1 * _CHUNK, _CHUNK)], swb)
            w1.wait()
            w2.wait()

            @pl.when(c0 + 2 < n_chunks)
            def _():
                pltpu.async_copy(a_hbm.at[srcv.at[c0 + 2]], ra0, sga0)
                pltpu.async_copy(b_hbm.at[dstv.at[c0 + 2]], rb0, sgb0)

            w3.wait()
            w4.wait()

            @pl.when(c1 + 2 < n_chunks)
            def _():
                pltpu.async_copy(a_hbm.at[srcv.at[c1 + 2]], ra1, sga1)
                pltpu.async_copy(b_hbm.at[dstv.at[c1 + 2]], rb1, sgb1)

    return pl.kernel(
        body,
        out_type=(jax.ShapeDtypeStruct((e_pad, _W), _f32),
                  jax.ShapeDtypeStruct((e_pad, _W), _f32)),
        mesh=_MESH,
        scratch_types=[
            pltpu.VMEM((n_chunks, _CHUNK), jnp.int32),
            pltpu.VMEM((n_chunks, _CHUNK), jnp.int32),
            pltpu.VMEM((_CHUNK, _W), _f32),
            pltpu.VMEM((_CHUNK, _W), _f32),
            pltpu.VMEM((_CHUNK, _W), _f32),
            pltpu.VMEM((_CHUNK, _W), _f32),
            pltpu.SemaphoreType.DMA,
            pltpu.SemaphoreType.DMA,
            pltpu.SemaphoreType.DMA,
            pltpu.SemaphoreType.DMA,
            pltpu.SemaphoreType.DMA,
        ],
    )


def _make_scatter(e_pad):
    per_worker = e_pad // _NW
    n_chunks = per_worker // _CHUNK

    def body(m_hbm, dst_hbm, out_hbm, dstv, mb0, mb1, accum,
             sl0, sl1, ss0, ss1):
        cid = lax.axis_index("c")
        sid = lax.axis_index("s")
        wid = sid * _NC + cid
        row0 = sid * _ROWS_PER_TILE
        base_c = wid * n_chunks
        base_e = wid * per_worker

        z = jnp.zeros((_L,), _f32)

        @pl.loop(0, _CHUNK)
        def _(i):
            r = mb0.at[i]
            for j in range(_W // _L):
                r[pl.ds(j * _L, _L)] = z

        @pl.loop(0, _ROWS_PER_TILE // _CHUNK)
        def _(k):
            pltpu.sync_copy(mb0, accum.at[pl.ds(row0 + k * _CHUNK, _CHUNK)])

        pltpu.sync_copy(dst_hbm.at[pl.ds(base_c, n_chunks)], dstv)
        plsc.subcore_barrier()

        pltpu.async_copy(m_hbm.at[pl.ds(base_e, _CHUNK)], mb0, sl0)
        pltpu.async_copy(m_hbm.at[pl.ds(base_e + _CHUNK, _CHUNK)], mb1, sl1)

        @pl.loop(0, n_chunks, step=2)
        def _(c0):
            c1 = c0 + 1
            _drain(m_hbm.at[pl.ds(base_e + c0 * _CHUNK, _CHUNK)], mb0, sl0)
            s0 = pltpu.async_copy(mb0, accum.at[dstv.at[c0]], ss0, add=True)
            _drain(m_hbm.at[pl.ds(base_e + c1 * _CHUNK, _CHUNK)], mb1, sl1)
            s1 = pltpu.async_copy(mb1, accum.at[dstv.at[c1]], ss1, add=True)
            s0.wait()

            @pl.when(c0 + 2 < n_chunks)
            def _():
                pltpu.async_copy(
                    m_hbm.at[pl.ds(base_e + (c0 + 2) * _CHUNK, _CHUNK)],
                    mb0, sl0)

            s1.wait()

            @pl.when(c1 + 2 < n_chunks)
            def _():
                pltpu.async_copy(
                    m_hbm.at[pl.ds(base_e + (c1 + 2) * _CHUNK, _CHUNK)],
                    mb1, sl1)

        plsc.subcore_barrier()

        pltpu.sync_copy(accum.at[pl.ds(row0, _ROWS_PER_TILE)],
                        out_hbm.at[cid, pl.ds(row0, _ROWS_PER_TILE)])

    return pl.kernel(
        body,
        out_type=jax.ShapeDtypeStruct((_NC, _N_PAD, _W), _f32),
        mesh=_MESH,
        scratch_types=[
            pltpu.VMEM((n_chunks, _CHUNK), jnp.int32),
            pltpu.VMEM((_CHUNK, _W), _f32),
            pltpu.VMEM((_CHUNK, _W), _f32),
            pltpu.VMEM_SHARED((_N_PAD, _W), _f32),
            pltpu.SemaphoreType.DMA,
            pltpu.SemaphoreType.DMA,
            pltpu.SemaphoreType.DMA,
            pltpu.SemaphoreType.DMA,
        ],
    )


_BLK_N = 2000
_BLK_NP = 1280


def _embed_body(u50_ref, w1_ref, b1_ref, w2_ref, b2_ref, x_ref):
    h = jnp.tanh(jnp.dot(u50_ref[...], w1_ref[...], precision=_HP) + b1_ref[...])
    x_ref[...] = jnp.tanh(jnp.dot(h, w2_ref[...], precision=_HP) + b2_ref[...])


def _embed(u50, w1, b1, w2, b2):
    return pl.pallas_call(
        _embed_body,
        grid=(_N // _BLK_N,),
        in_specs=[
            pl.BlockSpec((_BLK_N, 50), lambda i: (i, 0)),
            pl.BlockSpec((50, _HID), lambda i: (0, 0)),
            pl.BlockSpec((1, _HID), lambda i: (0, 0)),
            pl.BlockSpec((_HID, _HID), lambda i: (0, 0)),
            pl.BlockSpec((1, _HID), lambda i: (0, 0)),
        ],
        out_specs=pl.BlockSpec((_BLK_N, _HID), lambda i: (i, 0)),
        out_shape=jax.ShapeDtypeStruct((_N, _HID), _f32),
    )(u50, w1, b1, w2, b2)


def _pre_int_body(x66_ref, w1_ref, w2_ref, b_ref, t_ref):
    x66 = x66_ref[...]
    a = jnp.dot(x66, w1_ref[...], precision=_HP)
    bn = jnp.dot(x66, w2_ref[...], precision=_HP) + b_ref[...]
    t_ref[...] = jnp.concatenate([a, bn], axis=1)


def _pre_int(x66p, w1c, w2c, b):
    return pl.pallas_call(
        _pre_int_body,
        grid=(_N_PAD // _BLK_NP,),
        in_specs=[
            pl.BlockSpec((_BLK_NP, 66), lambda i: (i, 0)),
            pl.BlockSpec((66, _HID), lambda i: (0, 0)),
            pl.BlockSpec((66, _HID), lambda i: (0, 0)),
            pl.BlockSpec((1, _HID), lambda i: (0, 0)),
        ],
        out_specs=pl.BlockSpec((_BLK_NP, _W), lambda i: (i, 0)),
        out_shape=jax.ShapeDtypeStruct((_N_PAD, _W), _f32),
    )(x66p, w1c, w2c, b)


def _pre_ext_e_body(ef_ref, we_ref, te_ref):
    a = jnp.dot(ef_ref[...], we_ref[...], precision=_HP)
    te_ref[...] = jnp.concatenate([a, jnp.zeros((_NE, _HID), _f32)], axis=1)


def _pre_ext_e(ef18, wec):
    return pl.pallas_call(
        _pre_ext_e_body,
        out_shape=jax.ShapeDtypeStruct((_NE, _W), _f32),
    )(ef18, wec)


def _edge_body(g1_ref, g2_ref, w_ref, b_ref, m_ref):
    h = jnp.tanh(g1_ref[:, 0:_HID] + g2_ref[:, _HID:_W])
    m = jnp.tanh(jnp.dot(h, w_ref[...], precision=_HP) + b_ref[...])
    eblk = m.shape[0]
    tail = jnp.concatenate(
        [jnp.ones((eblk, 1), _f32), jnp.zeros((eblk, _HID - 1), _f32)], axis=1)
    m_ref[...] = jnp.concatenate([m, tail], axis=1)


def _edge_mlp(g1, g2, w, b, e_pad, eblk):
    return pl.pallas_call(
        _edge_body,
        grid=(e_pad // eblk,),
        in_specs=[
            pl.BlockSpec((eblk, _W), lambda i: (i, 0)),
            pl.BlockSpec((eblk, _W), lambda i: (i, 0)),
            pl.BlockSpec((_HID, _HID), lambda i: (0, 0)),
            pl.BlockSpec((1, _HID), lambda i: (0, 0)),
        ],
        out_specs=pl.BlockSpec((eblk, _W), lambda i: (i, 0)),
        out_shape=jax.ShapeDtypeStruct((e_pad, _W), _f32),
    )(g1, g2, w, b)


def _agg_from(part_ref):
    p = part_ref[0, :, 0:_HID] + part_ref[1, :, 0:_HID]
    count = part_ref[0, :, _HID:_HID + 1] + part_ref[1, :, _HID:_HID + 1]
    return p * (1.0 / jnp.maximum(count, 1.0))


_SPEC_X = pl.BlockSpec((_BLK_N, _HID), lambda i: (i, 0))
_SPEC_U = pl.BlockSpec((_BLK_N, _FEAT_M), lambda i: (i, 0))
_SPEC_PART = pl.BlockSpec((2, _BLK_N, _W), lambda i: (0, i, 0))
_SPEC_W64 = pl.BlockSpec((_HID, _HID), lambda i: (0, 0))
_SPEC_WU = pl.BlockSpec((_FEAT_M, _HID), lambda i: (0, 0))
_SPEC_B = pl.BlockSpec((1, _HID), lambda i: (0, 0))


def _upd_int_body(x_ref, u_ref, part_ref, wa_ref, wb_ref, wc_ref,
                  b1_ref, w2_ref, b2_ref, o_ref):
    agg = _agg_from(part_ref)
    h = jnp.tanh(jnp.dot(x_ref[...], wa_ref[...], precision=_HP)
                 + jnp.dot(agg, wb_ref[...], precision=_HP)
                 + jnp.dot(u_ref[...], wc_ref[...], precision=_HP)
                 + b1_ref[...])
    o_ref[...] = jnp.dot(h, w2_ref[...], precision=_HP) + b2_ref[...]


def _upd_int(x, u, part, wa, wb, wc, b1, w2, b2):
    return pl.pallas_call(
        _upd_int_body,
        grid=(_N // _BLK_N,),
        in_specs=[_SPEC_X, _SPEC_U, _SPEC_PART, _SPEC_W64, _SPEC_W64,
                  _SPEC_WU, _SPEC_B, _SPEC_W64, _SPEC_B],
        out_specs=_SPEC_X,
        out_shape=jax.ShapeDtypeStruct((_N, _HID), _f32),
    )(x, u, part, wa, wb, wc, b1, w2, b2)


def _upd_ext_body(x_ref, part_ref, wa_ref, wb_ref, b1_ref, w2_ref, b2_ref,
                  o_ref):
    agg = _agg_from(part_ref)
    h = jnp.tanh(jnp.dot(x_ref[...], wa_ref[...], precision=_HP)
                 + jnp.dot(agg, wb_ref[...], precision=_HP)
                 + b1_ref[...])
    o_ref[...] = jnp.dot(h, w2_ref[...], precision=_HP) + b2_ref[...]


def _upd_ext(x, part, wa, wb, b1, w2, b2):
    return pl.pallas_call(
        _upd_ext_body,
        grid=(_N // _BLK_N,),
        in_specs=[_SPEC_X, _SPEC_PART, _SPEC_W64, _SPEC_W64, _SPEC_B,
                  _SPEC_W64, _SPEC_B],
        out_specs=_SPEC_X,
        out_shape=jax.ShapeDtypeStruct((_N, _HID), _f32),
    )(x, part, wa, wb, b1, w2, b2)


def _out_body(x_ref, w1_ref, b1_ref, w2_ref, b2_ref, o_ref):
    h = jnp.tanh(jnp.dot(x_ref[...], w1_ref[...], precision=_HP) + b1_ref[...])
    o_ref[...] = jnp.dot(h, w2_ref[...], precision=_HP) + b2_ref[...]


def _out_head(x, w1, b1, w2, b2):
    return pl.pallas_call(
        _out_body,
        grid=(_N // _BLK_N,),
        in_specs=[_SPEC_X, _SPEC_W64, _SPEC_B,
                  pl.BlockSpec((_HID, _NOUT), lambda i: (0, 0)),
                  pl.BlockSpec((1, _NOUT), lambda i: (0, 0))],
        out_specs=pl.BlockSpec((_BLK_N, _NOUT), lambda i: (i, 0)),
        out_shape=jax.ShapeDtypeStruct((_N, _NOUT), _f32),
    )(x, w1, b1, w2, b2)


def _pad_edges(src, dst, e_pad, src_mod):
    e = src.shape[0]
    pi = jnp.arange(e_pad - e, dtype=jnp.int32)
    src = jnp.concatenate([src, pi % src_mod])
    dst = jnp.concatenate([dst, _N + pi % (_N_PAD - _N)])
    return src.reshape(-1, _CHUNK), dst.reshape(-1, _CHUNK)


def kernel(madis_x, madis_lon, madis_lat, edge_index, ex_lon, ex_lat, ex_x,
           edge_index_e2m, params):
    p = params
    u = madis_x.reshape(_N, _FEAT_M)
    pos = jnp.concatenate([madis_lon, madis_lat], axis=2).reshape(_N, 2)
    ei = (edge_index + (jnp.arange(_B) * _NM)[:, None, None]
          ).transpose(1, 0, 2).reshape(2, -1)
    exf = ex_x.reshape(_NE, _FEAT_E)
    ex_pos = jnp.concatenate([ex_lon[..., None], ex_lat[..., None]],
                             axis=2).reshape(_NE, 2)
    shift_e = jnp.stack([jnp.arange(_B) * _NES, jnp.arange(_B) * _NM],
                        axis=1)[..., None]
    ei_e = (edge_index_e2m + shift_e).transpose(1, 0, 2).reshape(2, -1)

    src1, dst1 = _pad_edges(ei[0], ei[1], _E1_PAD, _N)
    src2, dst2 = _pad_edges(ei_e[0], ei_e[1], _E2_PAD, _NE)

    gather1 = _make_gather(_E1_PAD)
    gather2 = _make_gather(_E2_PAD)
    scatter1 = _make_scatter(_E1_PAD)
    scatter2 = _make_scatter(_E2_PAD)

    ef18 = jnp.concatenate([exf, ex_pos], axis=1)

    u50 = jnp.concatenate([u, pos], axis=1)
    x = _embed(u50, p['emb1']['w'], p['emb1']['b'][None, :],
               p['emb2']['w'], p['emb2']['b'][None, :])

    zero66 = jnp.zeros((66, _HID), _f32)
    padrows = jnp.zeros((_N_PAD - _N, 66), _f32)

    def ext_layer(x, tag):
        w1 = p[tag + '_m1']['w']
        wec = jnp.concatenate([w1[0:16], w1[80:82]], axis=0)
        wxc = jnp.concatenate([w1[16:80], -w1[80:82]], axis=0)
        x66p = jnp.concatenate(
            [jnp.concatenate([x, pos], axis=1), padrows], axis=0)
        t_e = _pre_ext_e(ef18, wec)
        t_n = _pre_int(x66p, zero66, wxc, p[tag + '_m1']['b'][None, :])
        g1, g2 = gather2(t_e, t_n, src2, dst2)
        m = _edge_mlp(g1, g2, p[tag + '_m2']['w'],
                      p[tag + '_m2']['b'][None, :], _E2_PAD, 4096)
        part = scatter2(m, dst2)
        wu1 = p[tag + '_u1']['w']
        return _upd_ext(x, part, wu1[0:64], wu1[64:128],
                        p[tag + '_u1']['b'][None, :], p[tag + '_u2']['w'],
                        p[tag + '_u2']['b'][None, :])

    x = ext_layer(x, 'ex1')

    for i in range(_NPASS):
        lp = p['int'][i]
        w1 = lp['m1']['w']
        w1c = jnp.concatenate([w1[0:64], -w1[128:130]], axis=0)
        w2c = jnp.concatenate([w1[64:128], w1[128:130]], axis=0)
        x66p = jnp.concatenate(
            [jnp.concatenate([x, pos], axis=1), padrows], axis=0)
        t = _pre_int(x66p, w1c, w2c, lp['m1']['b'][None, :])
        g1, g2 = gather1(t, t, src1, dst1)
        m = _edge_mlp(g1, g2, lp['m2']['w'], lp['m2']['b'][None, :],
                      _E1_PAD, 8192)
        part = scatter1(m, dst1)
        wu1 = lp['u1']['w']
        x = _upd_int(x, u, part, wu1[0:64], wu1[64:128], wu1[128:176],
                     lp['u1']['b'][None, :], lp['u2']['w'],
                     lp['u2']['b'][None, :])

    x = ext_layer(x, 'ex2')

    out = _out_head(x, p['out1']['w'], p['out1']['b'][None, :],
                    p['out2']['w'], p['out2']['b'][None, :])
    return out.reshape(_B, _NM, _NOUT)


# speedup vs baseline: 1.1453x; 1.1453x over previous
import jax
import jax.numpy as jnp
from jax import lax
from jax.experimental import pallas as pl
from jax.experimental.pallas import tpu as pltpu
from jax.experimental.pallas import tpu_sc as plsc

_B, _NM, _NES = 4, 2500, 500
_FEAT_M = 48
_FEAT_E = 16
_HID, _NOUT, _NPASS = 64, 4, 3
_E_M2M, _E_E2M = 80000, 5000

_N = _B * _NM
_NE = _B * _NES
_W = 2 * _HID

_NC, _NS, _L = 2, 16, 16
_NW = _NC * _NS
_CHUNK = 128
_N_PAD = 10240
_ROWS_PER_TILE = _N_PAD // _NS

_EQ = _NW * _CHUNK * 8
_E1 = _B * _E_M2M
_E1_PAD = _EQ * ((_E1 + _EQ - 1) // _EQ)
_E2 = _B * _E_E2M
_E2_PAD = _EQ * ((_E2 + _EQ - 1) // _EQ)

_HP = jax.lax.Precision.HIGHEST
_f32 = jnp.float32
_bf16 = jnp.bfloat16

_MESH = plsc.VectorSubcoreMesh(core_axis_name="c", subcore_axis_name="s")


def _drain(src, dst, sem):
    pltpu.make_async_copy(src, dst, sem).wait()


def _make_gather(e_pad):
    per_worker = e_pad // _NW
    n_chunks = per_worker // _CHUNK
    assert n_chunks % 2 == 0 and (n_chunks % 8 == 0 or _NW * n_chunks % 8 == 0)

    def _compact_add(ra, rb, hb):
        @pl.loop(0, _CHUNK, step=4)
        def _(i0):
            for di in range(4):
                i = i0 + di
                for j in range(_HID // _L):
                    hb[i, pl.ds(j * _L, _L)] = (
                        ra[i, pl.ds(j * _L, _L)]
                        + rb[i, pl.ds(_HID + j * _L, _L)])

    def body(a_hbm, b_hbm, src_hbm, dst_hbm, g_hbm,
             srcv, dstv, ra0, rb0, ra1, rb1, hb0, hb1,
             sga0, sgb0, sga1, sgb1, swb):
        wid = lax.axis_index("s") * _NC + lax.axis_index("c")
        base_c = wid * n_chunks
        base_e = wid * per_worker
        pltpu.sync_copy(src_hbm.at[pl.ds(base_c, n_chunks)], srcv)
        pltpu.sync_copy(dst_hbm.at[pl.ds(base_c, n_chunks)], dstv)
        pltpu.async_copy(a_hbm.at[srcv.at[0]], ra0, sga0)
        pltpu.async_copy(b_hbm.at[dstv.at[0]], rb0, sgb0)
        pltpu.async_copy(a_hbm.at[srcv.at[1]], ra1, sga1)
        pltpu.async_copy(b_hbm.at[dstv.at[1]], rb1, sgb1)

        @pl.loop(0, n_chunks, step=2)
        def _(c0):
            c1 = c0 + 1
            _drain(a_hbm.at[srcv.at[c0]], ra0, sga0)
            _drain(b_hbm.at[dstv.at[c0]], rb0, sgb0)
            _compact_add(ra0, rb0, hb0)

            @pl.when(c0 + 2 < n_chunks)
            def _():
                pltpu.async_copy(a_hbm.at[srcv.at[c0 + 2]], ra0, sga0)
                pltpu.async_copy(b_hbm.at[dstv.at[c0 + 2]], rb0, sgb0)

            w0 = pltpu.async_copy(
                hb0, g_hbm.at[pl.ds(base_e + c0 * _CHUNK, _CHUNK)], swb)
            _drain(a_hbm.at[srcv.at[c1]], ra1, sga1)
            _drain(b_hbm.at[dstv.at[c1]], rb1, sgb1)
            _compact_add(ra1, rb1, hb1)

            @pl.when(c1 + 2 < n_chunks)
            def _():
                pltpu.async_copy(a_hbm.at[srcv.at[c1 + 2]], ra1, sga1)
                pltpu.async_copy(b_hbm.at[dstv.at[c1 + 2]], rb1, sgb1)

            w1 = pltpu.async_copy(
                hb1, g_hbm.at[pl.ds(base_e + c1 * _CHUNK, _CHUNK)], swb)
            w0.wait()
            w1.wait()

    return pl.kernel(
        body,
        out_type=jax.ShapeDtypeStruct((e_pad, _HID), _f32),
        mesh=_MESH,
        scratch_types=[
            pltpu.VMEM((n_chunks, _CHUNK), jnp.int32),
            pltpu.VMEM((n_chunks, _CHUNK), jnp.int32),
            pltpu.VMEM((_CHUNK, _W), _f32),
            pltpu.VMEM((_CHUNK, _W), _f32),
            pltpu.VMEM((_CHUNK, _W), _f32),
            pltpu.VMEM((_CHUNK, _W), _f32),
            pltpu.VMEM((_CHUNK, _HID), _f32),
            pltpu.VMEM((_CHUNK, _HID), _f32),
            pltpu.SemaphoreType.DMA,
            pltpu.SemaphoreType.DMA,
            pltpu.SemaphoreType.DMA,
            pltpu.SemaphoreType.DMA,
            pltpu.SemaphoreType.DMA,
        ],
    )


def _make_scatter(e_pad):
    per_worker = e_pad // _NW
    n_chunks = per_worker // _CHUNK

    def body(m_hbm, dst_hbm, out_hbm, dstv, mb0, mb1, accum,
             sl0, sl1, ss0, ss1):
        cid = lax.axis_index("c")
        sid = lax.axis_index("s")
        wid = sid * _NC + cid
        row0 = sid * _ROWS_PER_TILE
        base_c = wid * n_chunks
        base_e = wid * per_worker

        z = jnp.zeros((_L,), _f32)

        @pl.loop(0, _CHUNK)
        def _(i):
            r = mb0.at[i]
            for j in range(_W // _L):
                r[pl.ds(j * _L, _L)] = z

        @pl.loop(0, _ROWS_PER_TILE // _CHUNK)
        def _(k):
            pltpu.sync_copy(mb0, accum.at[pl.ds(row0 + k * _CHUNK, _CHUNK)])

        pltpu.sync_copy(dst_hbm.at[pl.ds(base_c, n_chunks)], dstv)
        plsc.subcore_barrier()

        pltpu.async_copy(m_hbm.at[pl.ds(base_e, _CHUNK)], mb0, sl0)
        pltpu.async_copy(m_hbm.at[pl.ds(base_e + _CHUNK, _CHUNK)], mb1, sl1)

        @pl.loop(0, n_chunks, step=2)
        def _(c0):
            c1 = c0 + 1
            _drain(m_hbm.at[pl.ds(base_e + c0 * _CHUNK, _CHUNK)], mb0, sl0)
            s0 = pltpu.async_copy(mb0, accum.at[dstv.at[c0]], ss0, add=True)
            _drain(m_hbm.at[pl.ds(base_e + c1 * _CHUNK, _CHUNK)], mb1, sl1)
            s1 = pltpu.async_copy(mb1, accum.at[dstv.at[c1]], ss1, add=True)
            s0.wait()

            @pl.when(c0 + 2 < n_chunks)
            def _():
                pltpu.async_copy(
                    m_hbm.at[pl.ds(base_e + (c0 + 2) * _CHUNK, _CHUNK)],
                    mb0, sl0)

            s1.wait()

            @pl.when(c1 + 2 < n_chunks)
            def _():
                pltpu.async_copy(
                    m_hbm.at[pl.ds(base_e + (c1 + 2) * _CHUNK, _CHUNK)],
                    mb1, sl1)

        plsc.subcore_barrier()

        pltpu.sync_copy(accum.at[pl.ds(row0, _ROWS_PER_TILE)],
                        out_hbm.at[cid, pl.ds(row0, _ROWS_PER_TILE)])

    return pl.kernel(
        body,
        out_type=jax.ShapeDtypeStruct((_NC, _N_PAD, _W), _f32),
        mesh=_MESH,
        scratch_types=[
            pltpu.VMEM((n_chunks, _CHUNK), jnp.int32),
            pltpu.VMEM((_CHUNK, _W), _f32),
            pltpu.VMEM((_CHUNK, _W), _f32),
            pltpu.VMEM_SHARED((_N_PAD, _W), _f32),
            pltpu.SemaphoreType.DMA,
            pltpu.SemaphoreType.DMA,
            pltpu.SemaphoreType.DMA,
            pltpu.SemaphoreType.DMA,
        ],
    )


_BLK_N = 2000
_BLK_NP = 1280


def _embed_body(u50_ref, w1_ref, b1_ref, w2_ref, b2_ref, x_ref):
    h = jnp.tanh(jnp.dot(u50_ref[...], w1_ref[...], precision=_HP) + b1_ref[...])
    x_ref[...] = jnp.tanh(jnp.dot(h, w2_ref[...], precision=_HP) + b2_ref[...])


def _embed(u50, w1, b1, w2, b2):
    return pl.pallas_call(
        _embed_body,
        grid=(_N // _BLK_N,),
        in_specs=[
            pl.BlockSpec((_BLK_N, 50), lambda i: (i, 0)),
            pl.BlockSpec((50, _HID), lambda i: (0, 0)),
            pl.BlockSpec((1, _HID), lambda i: (0, 0)),
            pl.BlockSpec((_HID, _HID), lambda i: (0, 0)),
            pl.BlockSpec((1, _HID), lambda i: (0, 0)),
        ],
        out_specs=pl.BlockSpec((_BLK_N, _HID), lambda i: (i, 0)),
        out_shape=jax.ShapeDtypeStruct((_N, _HID), _f32),
    )(u50, w1, b1, w2, b2)


def _pre_int_body(x66_ref, w1_ref, w2_ref, b_ref, t_ref):
    x66 = x66_ref[...]
    a = jnp.dot(x66, w1_ref[...], precision=_HP)
    bn = jnp.dot(x66, w2_ref[...], precision=_HP) + b_ref[...]
    t_ref[...] = jnp.concatenate([a, bn], axis=1)


def _pre_int(x66p, w1c, w2c, b):
    return pl.pallas_call(
        _pre_int_body,
        grid=(_N_PAD // _BLK_NP,),
        in_specs=[
            pl.BlockSpec((_BLK_NP, 66), lambda i: (i, 0)),
            pl.BlockSpec((66, _HID), lambda i: (0, 0)),
            pl.BlockSpec((66, _HID), lambda i: (0, 0)),
            pl.BlockSpec((1, _HID), lambda i: (0, 0)),
        ],
        out_specs=pl.BlockSpec((_BLK_NP, _W), lambda i: (i, 0)),
        out_shape=jax.ShapeDtypeStruct((_N_PAD, _W), _f32),
    )(x66p, w1c, w2c, b)


def _pre_ext_e_body(ef_ref, we_ref, te_ref):
    a = jnp.dot(ef_ref[...], we_ref[...], precision=_HP)
    te_ref[...] = jnp.concatenate([a, jnp.zeros((_NE, _HID), _f32)], axis=1)


def _pre_ext_e(ef18, wec):
    return pl.pallas_call(
        _pre_ext_e_body,
        out_shape=jax.ShapeDtypeStruct((_NE, _W), _f32),
    )(ef18, wec)


def _edge_body(g_ref, w_ref, b_ref, m_ref):
    h = jnp.tanh(g_ref[...])
    m = jnp.tanh(jnp.dot(h, w_ref[...], precision=_HP) + b_ref[...])
    eblk = m.shape[0]
    tail = jnp.concatenate(
        [jnp.ones((eblk, 1), _f32), jnp.zeros((eblk, _HID - 1), _f32)], axis=1)
    m_ref[...] = jnp.concatenate([m, tail], axis=1)


def _edge_mlp(g, w, b, e_pad, eblk):
    return pl.pallas_call(
        _edge_body,
        grid=(e_pad // eblk,),
        in_specs=[
            pl.BlockSpec((eblk, _HID), lambda i: (i, 0)),
            pl.BlockSpec((_HID, _HID), lambda i: (0, 0)),
            pl.BlockSpec((1, _HID), lambda i: (0, 0)),
        ],
        out_specs=pl.BlockSpec((eblk, _W), lambda i: (i, 0)),
        out_shape=jax.ShapeDtypeStruct((e_pad, _W), _f32),
    )(g, w, b)


def _agg_from(part_ref):
    p = part_ref[0, :, 0:_HID] + part_ref[1, :, 0:_HID]
    count = part_ref[0, :, _HID:_HID + 1] + part_ref[1, :, _HID:_HID + 1]
    return p * (1.0 / jnp.maximum(count, 1.0))


_SPEC_X = pl.BlockSpec((_BLK_N, _HID), lambda i: (i, 0))
_SPEC_U = pl.BlockSpec((_BLK_N, _FEAT_M), lambda i: (i, 0))
_SPEC_PART = pl.BlockSpec((2, _BLK_N, _W), lambda i: (0, i, 0))
_SPEC_W64 = pl.BlockSpec((_HID, _HID), lambda i: (0, 0))
_SPEC_WU = pl.BlockSpec((_FEAT_M, _HID), lambda i: (0, 0))
_SPEC_B = pl.BlockSpec((1, _HID), lambda i: (0, 0))


def _upd_int_body(x_ref, u_ref, part_ref, wa_ref, wb_ref, wc_ref,
                  b1_ref, w2_ref, b2_ref, o_ref):
    agg = _agg_from(part_ref)
    h = jnp.tanh(jnp.dot(x_ref[...], wa_ref[...], precision=_HP)
                 + jnp.dot(agg, wb_ref[...], precision=_HP)
                 + jnp.dot(u_ref[...], wc_ref[...], precision=_HP)
                 + b1_ref[...])
    o_ref[...] = jnp.dot(h, w2_ref[...], precision=_HP) + b2_ref[...]


def _upd_int(x, u, part, wa, wb, wc, b1, w2, b2):
    return pl.pallas_call(
        _upd_int_body,
        grid=(_N // _BLK_N,),
        in_specs=[_SPEC_X, _SPEC_U, _SPEC_PART, _SPEC_W64, _SPEC_W64,
                  _SPEC_WU, _SPEC_B, _SPEC_W64, _SPEC_B],
        out_specs=_SPEC_X,
        out_shape=jax.ShapeDtypeStruct((_N, _HID), _f32),
    )(x, u, part, wa, wb, wc, b1, w2, b2)


def _upd_ext_body(x_ref, part_ref, wa_ref, wb_ref, b1_ref, w2_ref, b2_ref,
                  o_ref):
    agg = _agg_from(part_ref)
    h = jnp.tanh(jnp.dot(x_ref[...], wa_ref[...], precision=_HP)
                 + jnp.dot(agg, wb_ref[...], precision=_HP)
                 + b1_ref[...])
    o_ref[...] = jnp.dot(h, w2_ref[...], precision=_HP) + b2_ref[...]


def _upd_ext(x, part, wa, wb, b1, w2, b2):
    return pl.pallas_call(
        _upd_ext_body,
        grid=(_N // _BLK_N,),
        in_specs=[_SPEC_X, _SPEC_PART, _SPEC_W64, _SPEC_W64, _SPEC_B,
                  _SPEC_W64, _SPEC_B],
        out_specs=_SPEC_X,
        out_shape=jax.ShapeDtypeStruct((_N, _HID), _f32),
    )(x, part, wa, wb, b1, w2, b2)


def _out_body(x_ref, w1_ref, b1_ref, w2_ref, b2_ref, o_ref):
    h = jnp.tanh(jnp.dot(x_ref[...], w1_ref[...], precision=_HP) + b1_ref[...])
    o_ref[...] = jnp.dot(h, w2_ref[...], precision=_HP) + b2_ref[...]


def _out_head(x, w1, b1, w2, b2):
    return pl.pallas_call(
        _out_body,
        grid=(_N // _BLK_N,),
        in_specs=[_SPEC_X, _SPEC_W64, _SPEC_B,
                  pl.BlockSpec((_HID, _NOUT), lambda i: (0, 0)),
                  pl.BlockSpec((1, _NOUT), lambda i: (0, 0))],
        out_specs=pl.BlockSpec((_BLK_N, _NOUT), lambda i: (i, 0)),
        out_shape=jax.ShapeDtypeStruct((_N, _NOUT), _f32),
    )(x, w1, b1, w2, b2)


def _pad_edges(src, dst, e_pad, src_mod):
    e = src.shape[0]
    pi = jnp.arange(e_pad - e, dtype=jnp.int32)
    src = jnp.concatenate([src, pi % src_mod])
    dst = jnp.concatenate([dst, _N + pi % (_N_PAD - _N)])
    return src.reshape(-1, _CHUNK), dst.reshape(-1, _CHUNK)


def kernel(madis_x, madis_lon, madis_lat, edge_index, ex_lon, ex_lat, ex_x,
           edge_index_e2m, params):
    p = params
    u = madis_x.reshape(_N, _FEAT_M)
    pos = jnp.concatenate([madis_lon, madis_lat], axis=2).reshape(_N, 2)
    ei = (edge_index + (jnp.arange(_B) * _NM)[:, None, None]
          ).transpose(1, 0, 2).reshape(2, -1)
    exf = ex_x.reshape(_NE, _FEAT_E)
    ex_pos = jnp.concatenate([ex_lon[..., None], ex_lat[..., None]],
                             axis=2).reshape(_NE, 2)
    shift_e = jnp.stack([jnp.arange(_B) * _NES, jnp.arange(_B) * _NM],
                        axis=1)[..., None]
    ei_e = (edge_index_e2m + shift_e).transpose(1, 0, 2).reshape(2, -1)

    src1, dst1 = _pad_edges(ei[0], ei[1], _E1_PAD, _N)
    src2, dst2 = _pad_edges(ei_e[0], ei_e[1], _E2_PAD, _NE)

    gather1 = _make_gather(_E1_PAD)
    gather2 = _make_gather(_E2_PAD)
    scatter1 = _make_scatter(_E1_PAD)
    scatter2 = _make_scatter(_E2_PAD)

    ef18 = jnp.concatenate([exf, ex_pos], axis=1)

    u50 = jnp.concatenate([u, pos], axis=1)
    x = _embed(u50, p['emb1']['w'], p['emb1']['b'][None, :],
               p['emb2']['w'], p['emb2']['b'][None, :])

    zero66 = jnp.zeros((66, _HID), _f32)
    padrows = jnp.zeros((_N_PAD - _N, 66), _f32)

    def ext_layer(x, tag):
        w1 = p[tag + '_m1']['w']
        wec = jnp.concatenate([w1[0:16], w1[80:82]], axis=0)
        wxc = jnp.concatenate([w1[16:80], -w1[80:82]], axis=0)
        x66p = jnp.concatenate(
            [jnp.concatenate([x, pos], axis=1), padrows], axis=0)
        t_e = _pre_ext_e(ef18, wec)
        t_n = _pre_int(x66p, zero66, wxc, p[tag + '_m1']['b'][None, :])
        g = gather2(t_e, t_n, src2, dst2)
        m = _edge_mlp(g, p[tag + '_m2']['w'],
                      p[tag + '_m2']['b'][None, :], _E2_PAD, 4096)
        part = scatter2(m, dst2)
        wu1 = p[tag + '_u1']['w']
        return _upd_ext(x, part, wu1[0:64], wu1[64:128],
                        p[tag + '_u1']['b'][None, :], p[tag + '_u2']['w'],
                        p[tag + '_u2']['b'][None, :])

    x = ext_layer(x, 'ex1')

    for i in range(_NPASS):
        lp = p['int'][i]
        w1 = lp['m1']['w']
        w1c = jnp.concatenate([w1[0:64], -w1[128:130]], axis=0)
        w2c = jnp.concatenate([w1[64:128], w1[128:130]], axis=0)
        x66p = jnp.concatenate(
            [jnp.concatenate([x, pos], axis=1), padrows], axis=0)
        t = _pre_int(x66p, w1c, w2c, lp['m1']['b'][None, :])
        g = gather1(t, t, src1, dst1)
        m = _edge_mlp(g, lp['m2']['w'], lp['m2']['b'][None, :],
                      _E1_PAD, 8192)
        part = scatter1(m, dst1)
        wu1 = lp['u1']['w']
        x = _upd_int(x, u, part, wu1[0:64], wu1[64:128], wu1[128:176],
                     lp['u1']['b'][None, :], lp['u2']['w'],
                     lp['u2']['b'][None, :])

    x = ext_layer(x, 'ex2')

    out = _out_head(x, p['out1']['w'], p['out1']['b'][None, :],
                    p['out2']['w'], p['out2']['b'][None, :])
    return out.reshape(_B, _NM, _NOUT)


# speedup vs baseline: 1.1463x; 1.0008x over previous
import jax
import jax.numpy as jnp
from jax import lax
from jax.experimental import pallas as pl
from jax.experimental.pallas import tpu as pltpu
from jax.experimental.pallas import tpu_sc as plsc

_B, _NM, _NES = 4, 2500, 500
_FEAT_M = 48
_FEAT_E = 16
_HID, _NOUT, _NPASS = 64, 4, 3
_E_M2M, _E_E2M = 80000, 5000

_N = _B * _NM
_NE = _B * _NES
_W = 2 * _HID

_NC, _NS, _L = 2, 16, 16
_NW = _NC * _NS
_CHUNK = 128
_N_PAD = 10240
_ROWS_PER_TILE = _N_PAD // _NS

_EQ = _NW * _CHUNK * 8
_E1 = _B * _E_M2M
_E1_PAD = _EQ * ((_E1 + _EQ - 1) // _EQ)
_E2 = _B * _E_E2M
_E2_PAD = _EQ * ((_E2 + _EQ - 1) // _EQ)

_HP = jax.lax.Precision.HIGHEST
_f32 = jnp.float32
_bf16 = jnp.bfloat16

_MESH = plsc.VectorSubcoreMesh(core_axis_name="c", subcore_axis_name="s")


def _drain(src, dst, sem):
    pltpu.make_async_copy(src, dst, sem).wait()


def _make_gather(e_pad):
    per_worker = e_pad // _NW
    n_chunks = per_worker // _CHUNK
    assert n_chunks % 2 == 0 and (n_chunks % 8 == 0 or _NW * n_chunks % 8 == 0)

    def _compact_add(ra, rb, hb):
        @pl.loop(0, _CHUNK, step=4)
        def _(i0):
            for di in range(4):
                i = i0 + di
                for j in range(_HID // _L):
                    hb[i, pl.ds(j * _L, _L)] = (
                        ra[i, pl.ds(j * _L, _L)]
                        + rb[i, pl.ds(_HID + j * _L, _L)])

    def body(a_hbm, b_hbm, src_hbm, dst_hbm, g_hbm,
             srcv, dstv, ra0, rb0, ra1, rb1, hb0, hb1,
             sga0, sgb0, sga1, sgb1, swb):
        wid = lax.axis_index("s") * _NC + lax.axis_index("c")
        base_c = wid * n_chunks
        base_e = wid * per_worker
        pltpu.sync_copy(src_hbm.at[pl.ds(base_c, n_chunks)], srcv)
        pltpu.sync_copy(dst_hbm.at[pl.ds(base_c, n_chunks)], dstv)
        pltpu.async_copy(a_hbm.at[srcv.at[0]], ra0, sga0)
        pltpu.async_copy(b_hbm.at[dstv.at[0]], rb0, sgb0)
        pltpu.async_copy(a_hbm.at[srcv.at[1]], ra1, sga1)
        pltpu.async_copy(b_hbm.at[dstv.at[1]], rb1, sgb1)

        @pl.loop(0, n_chunks, step=2)
        def _(c0):
            c1 = c0 + 1
            _drain(a_hbm.at[srcv.at[c0]], ra0, sga0)
            _drain(b_hbm.at[dstv.at[c0]], rb0, sgb0)
            _compact_add(ra0, rb0, hb0)

            @pl.when(c0 + 2 < n_chunks)
            def _():
                pltpu.async_copy(a_hbm.at[srcv.at[c0 + 2]], ra0, sga0)
                pltpu.async_copy(b_hbm.at[dstv.at[c0 + 2]], rb0, sgb0)

            w0 = pltpu.async_copy(
                hb0, g_hbm.at[pl.ds(base_e + c0 * _CHUNK, _CHUNK)], swb)
            _drain(a_hbm.at[srcv.at[c1]], ra1, sga1)
            _drain(b_hbm.at[dstv.at[c1]], rb1, sgb1)
            _compact_add(ra1, rb1, hb1)

            @pl.when(c1 + 2 < n_chunks)
            def _():
                pltpu.async_copy(a_hbm.at[srcv.at[c1 + 2]], ra1, sga1)
                pltpu.async_copy(b_hbm.at[dstv.at[c1 + 2]], rb1, sgb1)

            w1 = pltpu.async_copy(
                hb1, g_hbm.at[pl.ds(base_e + c1 * _CHUNK, _CHUNK)], swb)
            w0.wait()
            w1.wait()

    return pl.kernel(
        body,
        out_type=jax.ShapeDtypeStruct((e_pad, _HID), _f32),
        mesh=_MESH,
        scratch_types=[
            pltpu.VMEM((n_chunks, _CHUNK), jnp.int32),
            pltpu.VMEM((n_chunks, _CHUNK), jnp.int32),
            pltpu.VMEM((_CHUNK, _W), _f32),
            pltpu.VMEM((_CHUNK, _W), _f32),
            pltpu.VMEM((_CHUNK, _W), _f32),
            pltpu.VMEM((_CHUNK, _W), _f32),
            pltpu.VMEM((_CHUNK, _HID), _f32),
            pltpu.VMEM((_CHUNK, _HID), _f32),
            pltpu.SemaphoreType.DMA,
            pltpu.SemaphoreType.DMA,
            pltpu.SemaphoreType.DMA,
            pltpu.SemaphoreType.DMA,
            pltpu.SemaphoreType.DMA,
        ],
    )


def _make_scatter(e_pad):
    per_worker = e_pad // _NW
    n_chunks = per_worker // _CHUNK

    def body(m_hbm, dst_hbm, out_hbm, dstv, mb0, mb1, accum,
             sl0, sl1, ss0, ss1):
        cid = lax.axis_index("c")
        sid = lax.axis_index("s")
        wid = sid * _NC + cid
        row0 = sid * _ROWS_PER_TILE
        base_c = wid * n_chunks
        base_e = wid * per_worker

        z = jnp.zeros((_L,), _f32)

        @pl.loop(0, _CHUNK)
        def _(i):
            r = mb0.at[i]
            for j in range(_W // _L):
                r[pl.ds(j * _L, _L)] = z

        @pl.loop(0, _ROWS_PER_TILE // _CHUNK)
        def _(k):
            pltpu.sync_copy(mb0, accum.at[pl.ds(row0 + k * _CHUNK, _CHUNK)])

        pltpu.sync_copy(dst_hbm.at[pl.ds(base_c, n_chunks)], dstv)
        plsc.subcore_barrier()

        pltpu.async_copy(m_hbm.at[pl.ds(base_e, _CHUNK)], mb0, sl0)
        pltpu.async_copy(m_hbm.at[pl.ds(base_e + _CHUNK, _CHUNK)], mb1, sl1)

        @pl.loop(0, n_chunks, step=2)
        def _(c0):
            c1 = c0 + 1
            _drain(m_hbm.at[pl.ds(base_e + c0 * _CHUNK, _CHUNK)], mb0, sl0)
            s0 = pltpu.async_copy(mb0, accum.at[dstv.at[c0]], ss0, add=True)
            _drain(m_hbm.at[pl.ds(base_e + c1 * _CHUNK, _CHUNK)], mb1, sl1)
            s1 = pltpu.async_copy(mb1, accum.at[dstv.at[c1]], ss1, add=True)
            s0.wait()

            @pl.when(c0 + 2 < n_chunks)
            def _():
                pltpu.async_copy(
                    m_hbm.at[pl.ds(base_e + (c0 + 2) * _CHUNK, _CHUNK)],
                    mb0, sl0)

            s1.wait()

            @pl.when(c1 + 2 < n_chunks)
            def _():
                pltpu.async_copy(
                    m_hbm.at[pl.ds(base_e + (c1 + 2) * _CHUNK, _CHUNK)],
                    mb1, sl1)

        plsc.subcore_barrier()

        pltpu.sync_copy(accum.at[pl.ds(row0, _ROWS_PER_TILE)],
                        out_hbm.at[cid, pl.ds(row0, _ROWS_PER_TILE)])

    return pl.kernel(
        body,
        out_type=jax.ShapeDtypeStruct((_NC, _N_PAD, _W), _f32),
        mesh=_MESH,
        scratch_types=[
            pltpu.VMEM((n_chunks, _CHUNK), jnp.int32),
            pltpu.VMEM((_CHUNK, _W), _f32),
            pltpu.VMEM((_CHUNK, _W), _f32),
            pltpu.VMEM_SHARED((_N_PAD, _W), _f32),
            pltpu.SemaphoreType.DMA,
            pltpu.SemaphoreType.DMA,
            pltpu.SemaphoreType.DMA,
            pltpu.SemaphoreType.DMA,
        ],
    )


_BLK_N = 2000
_BLK_NP = 1280


def _pre_ext_e_body(ef_ref, we_ref, te_ref):
    a = jnp.dot(ef_ref[...], we_ref[...], precision=_HP)
    te_ref[...] = jnp.concatenate([a, jnp.zeros((_NE, _HID), _f32)], axis=1)


def _pre_ext_e(ef18, wec):
    return pl.pallas_call(
        _pre_ext_e_body,
        out_shape=jax.ShapeDtypeStruct((_NE, _W), _f32),
    )(ef18, wec)


def _edge_body(g_ref, w_ref, b_ref, m_ref):
    h = jnp.tanh(g_ref[...])
    m = jnp.tanh(jnp.dot(h, w_ref[...], precision=_HP) + b_ref[...])
    eblk = m.shape[0]
    tail = jnp.concatenate(
        [jnp.ones((eblk, 1), _f32), jnp.zeros((eblk, _HID - 1), _f32)], axis=1)
    m_ref[...] = jnp.concatenate([m, tail], axis=1)


def _edge_mlp(g, w, b, e_pad, eblk):
    return pl.pallas_call(
        _edge_body,
        grid=(e_pad // eblk,),
        in_specs=[
            pl.BlockSpec((eblk, _HID), lambda i: (i, 0)),
            pl.BlockSpec((_HID, _HID), lambda i: (0, 0)),
            pl.BlockSpec((1, _HID), lambda i: (0, 0)),
        ],
        out_specs=pl.BlockSpec((eblk, _W), lambda i: (i, 0)),
        out_shape=jax.ShapeDtypeStruct((e_pad, _W), _f32),
    )(g, w, b)


def _agg_from(part_ref):
    p = part_ref[0, :, 0:_HID] + part_ref[1, :, 0:_HID]
    count = part_ref[0, :, _HID:_HID + 1] + part_ref[1, :, _HID:_HID + 1]
    return p * (1.0 / jnp.maximum(count, 1.0))


def _table_from(xn, pos, w1c_ref, w2c_ref, bpre_ref):
    a = (jnp.dot(xn, w1c_ref[0:_HID], precision=_HP)
         + jnp.dot(pos, w1c_ref[_HID:_HID + 2], precision=_HP))
    bn = (jnp.dot(xn, w2c_ref[0:_HID], precision=_HP)
          + jnp.dot(pos, w2c_ref[_HID:_HID + 2], precision=_HP)
          + bpre_ref[...])
    return jnp.concatenate([a, bn], axis=1)


_SPEC_XP = pl.BlockSpec((_BLK_NP, _HID), lambda i: (i, 0))
_SPEC_UP = pl.BlockSpec((_BLK_NP, _FEAT_M), lambda i: (i, 0))
_SPEC_POS = pl.BlockSpec((_BLK_NP, 2), lambda i: (i, 0))
_SPEC_PARTP = pl.BlockSpec((2, _BLK_NP, _W), lambda i: (0, i, 0))
_SPEC_W64 = pl.BlockSpec((_HID, _HID), lambda i: (0, 0))
_SPEC_W66 = pl.BlockSpec((66, _HID), lambda i: (0, 0))
_SPEC_WU = pl.BlockSpec((_FEAT_M, _HID), lambda i: (0, 0))
_SPEC_B = pl.BlockSpec((1, _HID), lambda i: (0, 0))
_SPEC_TP = pl.BlockSpec((_BLK_NP, _W), lambda i: (i, 0))
_GRID_NP = (_N_PAD // _BLK_NP,)


def _embed_pre_body(u50_ref, we1_ref, be1_ref, we2_ref, be2_ref,
                    w2c_ref, bpre_ref, x_ref, t_ref):
    h = jnp.tanh(jnp.dot(u50_ref[...], we1_ref[...], precision=_HP)
                 + be1_ref[...])
    xn = jnp.tanh(jnp.dot(h, we2_ref[...], precision=_HP) + be2_ref[...])
    x_ref[...] = xn
    pos = u50_ref[:, _FEAT_M:_FEAT_M + 2]
    bn = (jnp.dot(xn, w2c_ref[0:_HID], precision=_HP)
          + jnp.dot(pos, w2c_ref[_HID:_HID + 2], precision=_HP)
          + bpre_ref[...])
    t_ref[...] = jnp.concatenate([jnp.zeros((_BLK_NP, _HID), _f32), bn],
                                 axis=1)


def _embed_pre(u50p, we1, be1, we2, be2, w2c, bpre):
    return pl.pallas_call(
        _embed_pre_body,
        grid=_GRID_NP,
        in_specs=[pl.BlockSpec((_BLK_NP, 50), lambda i: (i, 0)),
                  pl.BlockSpec((50, _HID), lambda i: (0, 0)),
                  _SPEC_B, _SPEC_W64, _SPEC_B, _SPEC_W66, _SPEC_B],
        out_specs=(_SPEC_XP, _SPEC_TP),
        out_shape=(jax.ShapeDtypeStruct((_N_PAD, _HID), _f32),
                   jax.ShapeDtypeStruct((_N_PAD, _W), _f32)),
    )(u50p, we1, be1, we2, be2, w2c, bpre)


def _upd_int_pre_body(x_ref, u_ref, pos_ref, part_ref, wa_ref, wb_ref, wc_ref,
                      b1_ref, w2_ref, b2_ref, w1c_ref, w2c_ref, bpre_ref,
                      x_out, t_ref):
    agg = _agg_from(part_ref)
    h = jnp.tanh(jnp.dot(x_ref[...], wa_ref[...], precision=_HP)
                 + jnp.dot(agg, wb_ref[...], precision=_HP)
                 + jnp.dot(u_ref[...], wc_ref[...], precision=_HP)
                 + b1_ref[...])
    xn = jnp.dot(h, w2_ref[...], precision=_HP) + b2_ref[...]
    x_out[...] = xn
    t_ref[...] = _table_from(xn, pos_ref[...], w1c_ref, w2c_ref, bpre_ref)


def _upd_int_pre(x, u, pos, part, wa, wb, wc, b1, w2, b2, w1c, w2c, bpre):
    return pl.pallas_call(
        _upd_int_pre_body,
        grid=_GRID_NP,
        in_specs=[_SPEC_XP, _SPEC_UP, _SPEC_POS, _SPEC_PARTP, _SPEC_W64,
                  _SPEC_W64, _SPEC_WU, _SPEC_B, _SPEC_W64, _SPEC_B,
                  _SPEC_W66, _SPEC_W66, _SPEC_B],
        out_specs=(_SPEC_XP, _SPEC_TP),
        out_shape=(jax.ShapeDtypeStruct((_N_PAD, _HID), _f32),
                   jax.ShapeDtypeStruct((_N_PAD, _W), _f32)),
    )(x, u, pos, part, wa, wb, wc, b1, w2, b2, w1c, w2c, bpre)


def _upd_ext_pre_body(x_ref, pos_ref, part_ref, wa_ref, wb_ref,
                      b1_ref, w2_ref, b2_ref, w1c_ref, w2c_ref, bpre_ref,
                      x_out, t_ref):
    agg = _agg_from(part_ref)
    h = jnp.tanh(jnp.dot(x_ref[...], wa_ref[...], precision=_HP)
                 + jnp.dot(agg, wb_ref[...], precision=_HP)
                 + b1_ref[...])
    xn = jnp.dot(h, w2_ref[...], precision=_HP) + b2_ref[...]
    x_out[...] = xn
    t_ref[...] = _table_from(xn, pos_ref[...], w1c_ref, w2c_ref, bpre_ref)


def _upd_ext_pre(x, pos, part, wa, wb, b1, w2, b2, w1c, w2c, bpre):
    return pl.pallas_call(
        _upd_ext_pre_body,
        grid=_GRID_NP,
        in_specs=[_SPEC_XP, _SPEC_POS, _SPEC_PARTP, _SPEC_W64, _SPEC_W64,
                  _SPEC_B, _SPEC_W64, _SPEC_B, _SPEC_W66, _SPEC_W66, _SPEC_B],
        out_specs=(_SPEC_XP, _SPEC_TP),
        out_shape=(jax.ShapeDtypeStruct((_N_PAD, _HID), _f32),
                   jax.ShapeDtypeStruct((_N_PAD, _W), _f32)),
    )(x, pos, part, wa, wb, b1, w2, b2, w1c, w2c, bpre)


def _upd_ext_out_body(x_ref, part_ref, wa_ref, wb_ref, b1_ref, w2_ref, b2_ref,
                      wo1_ref, bo1_ref, wo2_ref, bo2_ref, o_ref):
    agg = _agg_from(part_ref)
    h = jnp.tanh(jnp.dot(x_ref[...], wa_ref[...], precision=_HP)
                 + jnp.dot(agg, wb_ref[...], precision=_HP)
                 + b1_ref[...])
    xn = jnp.dot(h, w2_ref[...], precision=_HP) + b2_ref[...]
    ho = jnp.tanh(jnp.dot(xn, wo1_ref[...], precision=_HP) + bo1_ref[...])
    o_ref[...] = jnp.dot(ho, wo2_ref[...], precision=_HP) + bo2_ref[...]


def _upd_ext_out(x, part, wa, wb, b1, w2, b2, wo1, bo1, wo2, bo2):
    return pl.pallas_call(
        _upd_ext_out_body,
        grid=_GRID_NP,
        in_specs=[_SPEC_XP, _SPEC_PARTP, _SPEC_W64, _SPEC_W64, _SPEC_B,
                  _SPEC_W64, _SPEC_B, _SPEC_W64, _SPEC_B,
                  pl.BlockSpec((_HID, _NOUT), lambda i: (0, 0)),
                  pl.BlockSpec((1, _NOUT), lambda i: (0, 0))],
        out_specs=pl.BlockSpec((_BLK_NP, _NOUT), lambda i: (i, 0)),
        out_shape=jax.ShapeDtypeStruct((_N_PAD, _NOUT), _f32),
    )(x, part, wa, wb, b1, w2, b2, wo1, bo1, wo2, bo2)


def _pad_edges(src, dst, e_pad, src_mod):
    e = src.shape[0]
    pi = jnp.arange(e_pad - e, dtype=jnp.int32)
    src = jnp.concatenate([src, pi % src_mod])
    dst = jnp.concatenate([dst, _N + pi % (_N_PAD - _N)])
    return src.reshape(-1, _CHUNK), dst.reshape(-1, _CHUNK)


def kernel(madis_x, madis_lon, madis_lat, edge_index, ex_lon, ex_lat, ex_x,
           edge_index_e2m, params):
    p = params
    u = madis_x.reshape(_N, _FEAT_M)
    pos = jnp.concatenate([madis_lon, madis_lat], axis=2).reshape(_N, 2)
    ei = (edge_index + (jnp.arange(_B) * _NM)[:, None, None]
          ).transpose(1, 0, 2).reshape(2, -1)
    exf = ex_x.reshape(_NE, _FEAT_E)
    ex_pos = jnp.concatenate([ex_lon[..., None], ex_lat[..., None]],
                             axis=2).reshape(_NE, 2)
    shift_e = jnp.stack([jnp.arange(_B) * _NES, jnp.arange(_B) * _NM],
                        axis=1)[..., None]
    ei_e = (edge_index_e2m + shift_e).transpose(1, 0, 2).reshape(2, -1)

    src1, dst1 = _pad_edges(ei[0], ei[1], _E1_PAD, _N)
    src2, dst2 = _pad_edges(ei_e[0], ei_e[1], _E2_PAD, _NE)

    gather1 = _make_gather(_E1_PAD)
    gather2 = _make_gather(_E2_PAD)
    scatter1 = _make_scatter(_E1_PAD)
    scatter2 = _make_scatter(_E2_PAD)

    ef18 = jnp.concatenate([exf, ex_pos], axis=1)
    zero66 = jnp.zeros((66, _HID), _f32)
    npad = _N_PAD - _N
    u_p = jnp.pad(u, ((0, npad), (0, 0)))
    pos_p = jnp.pad(pos, ((0, npad), (0, 0)))
    u50p = jnp.concatenate([u_p, pos_p], axis=1)

    def ext_w(tag):
        w1 = p[tag + '_m1']['w']
        wec = jnp.concatenate([w1[0:16], w1[80:82]], axis=0)
        wxc = jnp.concatenate([w1[16:80], -w1[80:82]], axis=0)
        return wec, wxc, p[tag + '_m1']['b'][None, :]

    def int_w(i):
        w1 = p['int'][i]['m1']['w']
        w1c = jnp.concatenate([w1[0:64], -w1[128:130]], axis=0)
        w2c = jnp.concatenate([w1[64:128], w1[128:130]], axis=0)
        return w1c, w2c, p['int'][i]['m1']['b'][None, :]

    wec1, wxc1, bpre1 = ext_w('ex1')
    wec2, wxc2, bpre2 = ext_w('ex2')

    x, t_n = _embed_pre(u50p, p['emb1']['w'], p['emb1']['b'][None, :],
                        p['emb2']['w'], p['emb2']['b'][None, :], wxc1, bpre1)

    def msg_round(gather, scatter, t_a, t_b, src, dst, m2, e_pad, eblk):
        g = gather(t_a, t_b, src, dst)
        m = _edge_mlp(g, m2['w'], m2['b'][None, :], e_pad, eblk)
        return scatter(m, dst)

    t_e = _pre_ext_e(ef18, wec1)
    part = msg_round(gather2, scatter2, t_e, t_n, src2, dst2, p['ex1_m2'],
                     _E2_PAD, 4096)
    wu1 = p['ex1_u1']['w']
    w1c, w2c, bpre = int_w(0)
    x, t = _upd_ext_pre(x, pos_p, part, wu1[0:64], wu1[64:128],
                        p['ex1_u1']['b'][None, :], p['ex1_u2']['w'],
                        p['ex1_u2']['b'][None, :], w1c, w2c, bpre)

    for i in range(_NPASS):
        part = msg_round(gather1, scatter1, t, t, src1, dst1, p['int'][i]['m2'],
                         _E1_PAD, 8192)
        lp = p['int'][i]
        wu1 = lp['u1']['w']
        if i + 1 < _NPASS:
            w1c, w2c, bpre = int_w(i + 1)
        else:
            w1c, w2c, bpre = zero66, wxc2, bpre2
        x, t = _upd_int_pre(x, u_p, pos_p, part, wu1[0:64], wu1[64:128],
                            wu1[128:176], lp['u1']['b'][None, :],
                            lp['u2']['w'], lp['u2']['b'][None, :],
                            w1c, w2c, bpre)

    t_e = _pre_ext_e(ef18, wec2)
    part = msg_round(gather2, scatter2, t_e, t, src2, dst2, p['ex2_m2'],
                     _E2_PAD, 4096)
    wu1 = p['ex2_u1']['w']
    out = _upd_ext_out(x, part, wu1[0:64], wu1[64:128],
                       p['ex2_u1']['b'][None, :], p['ex2_u2']['w'],
                       p['ex2_u2']['b'][None, :],
                       p['out1']['w'], p['out1']['b'][None, :],
                       p['out2']['w'], p['out2']['b'][None, :])
    return out[0:_N].reshape(_B, _NM, _NOUT)


# speedup vs baseline: 1.2231x; 1.0670x over previous
import jax
import jax.numpy as jnp
from jax import lax
from jax.experimental import pallas as pl
from jax.experimental.pallas import tpu as pltpu
from jax.experimental.pallas import tpu_sc as plsc

_B, _NM, _NES = 4, 2500, 500
_FEAT_M = 48
_FEAT_E = 16
_HID, _NOUT, _NPASS = 64, 4, 3
_E_M2M, _E_E2M = 80000, 5000

_N = _B * _NM
_NE = _B * _NES
_W = 2 * _HID

_NC, _NS, _L = 2, 16, 16
_NW = _NC * _NS
_CHUNK = 128
_N_PAD = 10240
_ROWS_PER_TILE = _N_PAD // _NS

_EQ = _NW * _CHUNK * 8
_E1 = _B * _E_M2M
_E1_PAD = _EQ * ((_E1 + _EQ - 1) // _EQ)
_E2 = _B * _E_E2M
_E2_PAD = _EQ * ((_E2 + _EQ - 1) // _EQ)

_HP = jax.lax.Precision.HIGHEST
_f32 = jnp.float32
_bf16 = jnp.bfloat16

_MESH = plsc.VectorSubcoreMesh(core_axis_name="c", subcore_axis_name="s")


def _drain(src, dst, sem):
    pltpu.make_async_copy(src, dst, sem).wait()


def _make_gather(e_pad):
    per_worker = e_pad // _NW
    n_chunks = per_worker // _CHUNK
    assert n_chunks % 2 == 0 and (n_chunks % 8 == 0 or _NW * n_chunks % 8 == 0)

    def _compact_add(ra, rb, hb):
        @pl.loop(0, _CHUNK, step=4)
        def _(i0):
            for di in range(4):
                i = i0 + di
                for j in range(_HID // _L):
                    hb[i, pl.ds(j * _L, _L)] = (
                        ra[i, pl.ds(j * _L, _L)]
                        + rb[i, pl.ds(_HID + j * _L, _L)])

    def body(a_hbm, b_hbm, src_hbm, dst_hbm, g_hbm,
             srcv, dstv, ra0, rb0, ra1, rb1, hb0, hb1,
             sga0, sgb0, sga1, sgb1, swb):
        wid = lax.axis_index("s") * _NC + lax.axis_index("c")
        base_c = wid * n_chunks
        base_e = wid * per_worker
        pltpu.sync_copy(src_hbm.at[pl.ds(base_c, n_chunks)], srcv)
        pltpu.sync_copy(dst_hbm.at[pl.ds(base_c, n_chunks)], dstv)
        pltpu.async_copy(a_hbm.at[srcv.at[0]], ra0, sga0)
        pltpu.async_copy(b_hbm.at[dstv.at[0]], rb0, sgb0)
        pltpu.async_copy(a_hbm.at[srcv.at[1]], ra1, sga1)
        pltpu.async_copy(b_hbm.at[dstv.at[1]], rb1, sgb1)

        @pl.loop(0, n_chunks, step=2)
        def _(c0):
            c1 = c0 + 1
            _drain(a_hbm.at[srcv.at[c0]], ra0, sga0)
            _drain(b_hbm.at[dstv.at[c0]], rb0, sgb0)
            _compact_add(ra0, rb0, hb0)

            @pl.when(c0 + 2 < n_chunks)
            def _():
                pltpu.async_copy(a_hbm.at[srcv.at[c0 + 2]], ra0, sga0)
                pltpu.async_copy(b_hbm.at[dstv.at[c0 + 2]], rb0, sgb0)

            w0 = pltpu.async_copy(
                hb0, g_hbm.at[pl.ds(base_e + c0 * _CHUNK, _CHUNK)], swb)
            _drain(a_hbm.at[srcv.at[c1]], ra1, sga1)
            _drain(b_hbm.at[dstv.at[c1]], rb1, sgb1)
            _compact_add(ra1, rb1, hb1)

            @pl.when(c1 + 2 < n_chunks)
            def _():
                pltpu.async_copy(a_hbm.at[srcv.at[c1 + 2]], ra1, sga1)
                pltpu.async_copy(b_hbm.at[dstv.at[c1 + 2]], rb1, sgb1)

            w1 = pltpu.async_copy(
                hb1, g_hbm.at[pl.ds(base_e + c1 * _CHUNK, _CHUNK)], swb)
            w0.wait()
            w1.wait()

    return pl.kernel(
        body,
        out_type=jax.ShapeDtypeStruct((e_pad, _HID), _f32),
        mesh=_MESH,
        scratch_types=[
            pltpu.VMEM((n_chunks, _CHUNK), jnp.int32),
            pltpu.VMEM((n_chunks, _CHUNK), jnp.int32),
            pltpu.VMEM((_CHUNK, _W), _f32),
            pltpu.VMEM((_CHUNK, _W), _f32),
            pltpu.VMEM((_CHUNK, _W), _f32),
            pltpu.VMEM((_CHUNK, _W), _f32),
            pltpu.VMEM((_CHUNK, _HID), _f32),
            pltpu.VMEM((_CHUNK, _HID), _f32),
            pltpu.SemaphoreType.DMA,
            pltpu.SemaphoreType.DMA,
            pltpu.SemaphoreType.DMA,
            pltpu.SemaphoreType.DMA,
            pltpu.SemaphoreType.DMA,
        ],
    )


def _make_scatter(e_pad):
    per_worker = e_pad // _NW
    n_chunks = per_worker // _CHUNK

    def body(m_hbm, dst_hbm, out_hbm, dstv, mb0, mb1, accum,
             sl0, sl1, ss0, ss1):
        cid = lax.axis_index("c")
        sid = lax.axis_index("s")
        wid = sid * _NC + cid
        row0 = sid * _ROWS_PER_TILE
        base_c = wid * n_chunks
        base_e = wid * per_worker

        z = jnp.zeros((_L,), _f32)

        @pl.loop(0, _CHUNK)
        def _(i):
            r = mb0.at[i]
            for j in range(_W // _L):
                r[pl.ds(j * _L, _L)] = z

        @pl.loop(0, _ROWS_PER_TILE // _CHUNK)
        def _(k):
            pltpu.sync_copy(mb0, accum.at[pl.ds(row0 + k * _CHUNK, _CHUNK)])

        pltpu.sync_copy(dst_hbm.at[pl.ds(base_c, n_chunks)], dstv)
        plsc.subcore_barrier()

        pltpu.async_copy(m_hbm.at[pl.ds(base_e, _CHUNK)], mb0, sl0)
        pltpu.async_copy(m_hbm.at[pl.ds(base_e + _CHUNK, _CHUNK)], mb1, sl1)

        @pl.loop(0, n_chunks, step=2)
        def _(c0):
            c1 = c0 + 1
            _drain(m_hbm.at[pl.ds(base_e + c0 * _CHUNK, _CHUNK)], mb0, sl0)
            s0 = pltpu.async_copy(mb0, accum.at[dstv.at[c0]], ss0, add=True)
            _drain(m_hbm.at[pl.ds(base_e + c1 * _CHUNK, _CHUNK)], mb1, sl1)
            s1 = pltpu.async_copy(mb1, accum.at[dstv.at[c1]], ss1, add=True)
            s0.wait()

            @pl.when(c0 + 2 < n_chunks)
            def _():
                pltpu.async_copy(
                    m_hbm.at[pl.ds(base_e + (c0 + 2) * _CHUNK, _CHUNK)],
                    mb0, sl0)

            s1.wait()

            @pl.when(c1 + 2 < n_chunks)
            def _():
                pltpu.async_copy(
                    m_hbm.at[pl.ds(base_e + (c1 + 2) * _CHUNK, _CHUNK)],
                    mb1, sl1)

        plsc.subcore_barrier()

        pltpu.sync_copy(accum.at[pl.ds(row0, _ROWS_PER_TILE)],
                        out_hbm.at[cid, pl.ds(row0, _ROWS_PER_TILE)])

    return pl.kernel(
        body,
        out_type=jax.ShapeDtypeStruct((_NC, _N_PAD, _W), _f32),
        mesh=_MESH,
        scratch_types=[
            pltpu.VMEM((n_chunks, _CHUNK), jnp.int32),
            pltpu.VMEM((_CHUNK, _W), _f32),
            pltpu.VMEM((_CHUNK, _W), _f32),
            pltpu.VMEM_SHARED((_N_PAD, _W), _f32),
            pltpu.SemaphoreType.DMA,
            pltpu.SemaphoreType.DMA,
            pltpu.SemaphoreType.DMA,
            pltpu.SemaphoreType.DMA,
        ],
    )


_BLK_N = 2000
_BLK_NP = 1280


def _pre_ext_e_body(ef_ref, we_ref, te_ref):
    a = jnp.dot(ef_ref[...], we_ref[...], precision=_HP)
    te_ref[...] = jnp.concatenate([a, jnp.zeros((_NE, _HID), _f32)], axis=1)


def _pre_ext_e(ef18, wec):
    return pl.pallas_call(
        _pre_ext_e_body,
        out_shape=jax.ShapeDtypeStruct((_NE, _W), _f32),
    )(ef18, wec)


def _edge_body(g_ref, w_ref, b_ref, m_ref):
    h = jnp.tanh(g_ref[...])
    m = jnp.tanh(jnp.dot(h, w_ref[...], precision=_HP) + b_ref[...])
    eblk = m.shape[0]
    tail = jnp.concatenate(
        [jnp.ones((eblk, 1), _f32), jnp.zeros((eblk, _HID - 1), _f32)], axis=1)
    m_ref[...] = jnp.concatenate([m, tail], axis=1)


def _edge_mlp(g, w, b, e_pad, eblk):
    return pl.pallas_call(
        _edge_body,
        grid=(e_pad // eblk,),
        in_specs=[
            pl.BlockSpec((eblk, _HID), lambda i: (i, 0)),
            pl.BlockSpec((_HID, _HID), lambda i: (0, 0)),
            pl.BlockSpec((1, _HID), lambda i: (0, 0)),
        ],
        out_specs=pl.BlockSpec((eblk, _W), lambda i: (i, 0)),
        out_shape=jax.ShapeDtypeStruct((e_pad, _W), _f32),
    )(g, w, b)


def _agg_from(pa_ref, pb_ref):
    p = (pa_ref[0, :, 0:_HID] + pa_ref[1, :, 0:_HID]
         + pb_ref[0, :, 0:_HID] + pb_ref[1, :, 0:_HID])
    count = (pa_ref[0, :, _HID:_HID + 1] + pa_ref[1, :, _HID:_HID + 1]
             + pb_ref[0, :, _HID:_HID + 1] + pb_ref[1, :, _HID:_HID + 1])
    return p * (1.0 / jnp.maximum(count, 1.0))


def _table_from(xn, pos, w1c_ref, w2c_ref, bpre_ref):
    a = (jnp.dot(xn, w1c_ref[0:_HID], precision=_HP)
         + jnp.dot(pos, w1c_ref[_HID:_HID + 2], precision=_HP))
    bn = (jnp.dot(xn, w2c_ref[0:_HID], precision=_HP)
          + jnp.dot(pos, w2c_ref[_HID:_HID + 2], precision=_HP)
          + bpre_ref[...])
    return jnp.concatenate([a, bn], axis=1)


_SPEC_XP = pl.BlockSpec((_BLK_NP, _HID), lambda i: (i, 0))
_SPEC_UP = pl.BlockSpec((_BLK_NP, _FEAT_M), lambda i: (i, 0))
_SPEC_POS = pl.BlockSpec((_BLK_NP, 2), lambda i: (i, 0))
_SPEC_PARTP = pl.BlockSpec((2, _BLK_NP, _W), lambda i: (0, i, 0))
_SPEC_W64 = pl.BlockSpec((_HID, _HID), lambda i: (0, 0))
_SPEC_W66 = pl.BlockSpec((66, _HID), lambda i: (0, 0))
_SPEC_WU = pl.BlockSpec((_FEAT_M, _HID), lambda i: (0, 0))
_SPEC_B = pl.BlockSpec((1, _HID), lambda i: (0, 0))
_SPEC_TP = pl.BlockSpec((_BLK_NP, _W), lambda i: (i, 0))
_GRID_NP = (_N_PAD // _BLK_NP,)


def _embed_pre_body(u50_ref, we1_ref, be1_ref, we2_ref, be2_ref,
                    w2c_ref, bpre_ref, x_ref, t_ref):
    h = jnp.tanh(jnp.dot(u50_ref[...], we1_ref[...], precision=_HP)
                 + be1_ref[...])
    xn = jnp.tanh(jnp.dot(h, we2_ref[...], precision=_HP) + be2_ref[...])
    x_ref[...] = xn
    pos = u50_ref[:, _FEAT_M:_FEAT_M + 2]
    bn = (jnp.dot(xn, w2c_ref[0:_HID], precision=_HP)
          + jnp.dot(pos, w2c_ref[_HID:_HID + 2], precision=_HP)
          + bpre_ref[...])
    t_ref[...] = jnp.concatenate([jnp.zeros((_BLK_NP, _HID), _f32), bn],
                                 axis=1)


def _embed_pre(u50p, we1, be1, we2, be2, w2c, bpre):
    return pl.pallas_call(
        _embed_pre_body,
        grid=_GRID_NP,
        in_specs=[pl.BlockSpec((_BLK_NP, 50), lambda i: (i, 0)),
                  pl.BlockSpec((50, _HID), lambda i: (0, 0)),
                  _SPEC_B, _SPEC_W64, _SPEC_B, _SPEC_W66, _SPEC_B],
        out_specs=(_SPEC_XP, _SPEC_TP),
        out_shape=(jax.ShapeDtypeStruct((_N_PAD, _HID), _f32),
                   jax.ShapeDtypeStruct((_N_PAD, _W), _f32)),
    )(u50p, we1, be1, we2, be2, w2c, bpre)


def _upd_int_pre_body(x_ref, u_ref, pos_ref, pa_ref, pb_ref, wa_ref, wb_ref,
                      wc_ref, b1_ref, w2_ref, b2_ref, w1c_ref, w2c_ref,
                      bpre_ref, x_out, t_ref):
    agg = _agg_from(pa_ref, pb_ref)
    h = jnp.tanh(jnp.dot(x_ref[...], wa_ref[...], precision=_HP)
                 + jnp.dot(agg, wb_ref[...], precision=_HP)
                 + jnp.dot(u_ref[...], wc_ref[...], precision=_HP)
                 + b1_ref[...])
    xn = jnp.dot(h, w2_ref[...], precision=_HP) + b2_ref[...]
    x_out[...] = xn
    t_ref[...] = _table_from(xn, pos_ref[...], w1c_ref, w2c_ref, bpre_ref)


def _upd_int_pre(x, u, pos, pa, pb, wa, wb, wc, b1, w2, b2, w1c, w2c, bpre):
    return pl.pallas_call(
        _upd_int_pre_body,
        grid=_GRID_NP,
        in_specs=[_SPEC_XP, _SPEC_UP, _SPEC_POS, _SPEC_PARTP, _SPEC_PARTP,
                  _SPEC_W64, _SPEC_W64, _SPEC_WU, _SPEC_B, _SPEC_W64, _SPEC_B,
                  _SPEC_W66, _SPEC_W66, _SPEC_B],
        out_specs=(_SPEC_XP, _SPEC_TP),
        out_shape=(jax.ShapeDtypeStruct((_N_PAD, _HID), _f32),
                   jax.ShapeDtypeStruct((_N_PAD, _W), _f32)),
    )(x, u, pos, pa, pb, wa, wb, wc, b1, w2, b2, w1c, w2c, bpre)


def _upd_ext_pre_body(x_ref, pos_ref, pa_ref, pb_ref, wa_ref, wb_ref,
                      b1_ref, w2_ref, b2_ref, w1c_ref, w2c_ref, bpre_ref,
                      x_out, t_ref):
    agg = _agg_from(pa_ref, pb_ref)
    h = jnp.tanh(jnp.dot(x_ref[...], wa_ref[...], precision=_HP)
                 + jnp.dot(agg, wb_ref[...], precision=_HP)
                 + b1_ref[...])
    xn = jnp.dot(h, w2_ref[...], precision=_HP) + b2_ref[...]
    x_out[...] = xn
    t_ref[...] = _table_from(xn, pos_ref[...], w1c_ref, w2c_ref, bpre_ref)


def _upd_ext_pre(x, pos, pa, pb, wa, wb, b1, w2, b2, w1c, w2c, bpre):
    return pl.pallas_call(
        _upd_ext_pre_body,
        grid=_GRID_NP,
        in_specs=[_SPEC_XP, _SPEC_POS, _SPEC_PARTP, _SPEC_PARTP, _SPEC_W64,
                  _SPEC_W64, _SPEC_B, _SPEC_W64, _SPEC_B, _SPEC_W66,
                  _SPEC_W66, _SPEC_B],
        out_specs=(_SPEC_XP, _SPEC_TP),
        out_shape=(jax.ShapeDtypeStruct((_N_PAD, _HID), _f32),
                   jax.ShapeDtypeStruct((_N_PAD, _W), _f32)),
    )(x, pos, pa, pb, wa, wb, b1, w2, b2, w1c, w2c, bpre)


def _upd_ext_out_body(x_ref, pa_ref, pb_ref, wa_ref, wb_ref, b1_ref, w2_ref,
                      b2_ref, wo1_ref, bo1_ref, wo2_ref, bo2_ref, o_ref):
    agg = _agg_from(pa_ref, pb_ref)
    h = jnp.tanh(jnp.dot(x_ref[...], wa_ref[...], precision=_HP)
                 + jnp.dot(agg, wb_ref[...], precision=_HP)
                 + b1_ref[...])
    xn = jnp.dot(h, w2_ref[...], precision=_HP) + b2_ref[...]
    ho = jnp.tanh(jnp.dot(xn, wo1_ref[...], precision=_HP) + bo1_ref[...])
    o_ref[...] = jnp.dot(ho, wo2_ref[...], precision=_HP) + bo2_ref[...]


def _upd_ext_out(x, pa, pb, wa, wb, b1, w2, b2, wo1, bo1, wo2, bo2):
    return pl.pallas_call(
        _upd_ext_out_body,
        grid=_GRID_NP,
        in_specs=[_SPEC_XP, _SPEC_PARTP, _SPEC_PARTP, _SPEC_W64, _SPEC_W64,
                  _SPEC_B,
                  _SPEC_W64, _SPEC_B, _SPEC_W64, _SPEC_B,
                  pl.BlockSpec((_HID, _NOUT), lambda i: (0, 0)),
                  pl.BlockSpec((1, _NOUT), lambda i: (0, 0))],
        out_specs=pl.BlockSpec((_BLK_NP, _NOUT), lambda i: (i, 0)),
        out_shape=jax.ShapeDtypeStruct((_N_PAD, _NOUT), _f32),
    )(x, pa, pb, wa, wb, b1, w2, b2, wo1, bo1, wo2, bo2)


def _pad_edges(src, dst, e_pad, src_mod):
    e = src.shape[0]
    pi = jnp.arange(e_pad - e, dtype=jnp.int32)
    src = jnp.concatenate([src, pi % src_mod])
    dst = jnp.concatenate([dst, _N + pi % (_N_PAD - _N)])
    return src.reshape(-1, _CHUNK), dst.reshape(-1, _CHUNK)


def kernel(madis_x, madis_lon, madis_lat, edge_index, ex_lon, ex_lat, ex_x,
           edge_index_e2m, params):
    p = params
    u = madis_x.reshape(_N, _FEAT_M)
    pos = jnp.concatenate([madis_lon, madis_lat], axis=2).reshape(_N, 2)
    ei = (edge_index + (jnp.arange(_B) * _NM)[:, None, None]
          ).transpose(1, 0, 2).reshape(2, -1)
    exf = ex_x.reshape(_NE, _FEAT_E)
    ex_pos = jnp.concatenate([ex_lon[..., None], ex_lat[..., None]],
                             axis=2).reshape(_NE, 2)
    shift_e = jnp.stack([jnp.arange(_B) * _NES, jnp.arange(_B) * _NM],
                        axis=1)[..., None]
    ei_e = (edge_index_e2m + shift_e).transpose(1, 0, 2).reshape(2, -1)

    src1, dst1 = _pad_edges(ei[0], ei[1], _E1_PAD, _N)
    src2, dst2 = _pad_edges(ei_e[0], ei_e[1], _E2_PAD, _NE)

    gather1 = _make_gather(_E1_PAD // 2)
    gather2 = _make_gather(_E2_PAD)
    scatter1 = _make_scatter(_E1_PAD // 2)
    scatter2 = _make_scatter(_E2_PAD)
    hc1 = _E1_PAD // 2 // _CHUNK

    ef18 = jnp.concatenate([exf, ex_pos], axis=1)
    zero66 = jnp.zeros((66, _HID), _f32)
    npad = _N_PAD - _N
    u_p = jnp.pad(u, ((0, npad), (0, 0)))
    pos_p = jnp.pad(pos, ((0, npad), (0, 0)))
    u50p = jnp.concatenate([u_p, pos_p], axis=1)

    def ext_w(tag):
        w1 = p[tag + '_m1']['w']
        wec = jnp.concatenate([w1[0:16], w1[80:82]], axis=0)
        wxc = jnp.concatenate([w1[16:80], -w1[80:82]], axis=0)
        return wec, wxc, p[tag + '_m1']['b'][None, :]

    def int_w(i):
        w1 = p['int'][i]['m1']['w']
        w1c = jnp.concatenate([w1[0:64], -w1[128:130]], axis=0)
        w2c = jnp.concatenate([w1[64:128], w1[128:130]], axis=0)
        return w1c, w2c, p['int'][i]['m1']['b'][None, :]

    wec1, wxc1, bpre1 = ext_w('ex1')
    wec2, wxc2, bpre2 = ext_w('ex2')

    x, t_n = _embed_pre(u50p, p['emb1']['w'], p['emb1']['b'][None, :],
                        p['emb2']['w'], p['emb2']['b'][None, :], wxc1, bpre1)

    def msg_round(gather, scatter, t_a, t_b, src, dst, m2, e_pad, eblk):
        g = gather(t_a, t_b, src, dst)
        m = _edge_mlp(g, m2['w'], m2['b'][None, :], e_pad, eblk)
        return scatter(m, dst)

    def msg_round_split(t, m2):
        pa = msg_round(gather1, scatter1, t, t, src1[:hc1], dst1[:hc1],
                       m2, _E1_PAD // 2, 8192)
        pb = msg_round(gather1, scatter1, t, t, src1[hc1:], dst1[hc1:],
                       m2, _E1_PAD // 2, 8192)
        return pa, pb

    t_e = _pre_ext_e(ef18, wec1)
    part = msg_round(gather2, scatter2, t_e, t_n, src2, dst2, p['ex1_m2'],
                     _E2_PAD, 4096)
    wu1 = p['ex1_u1']['w']
    w1c, w2c, bpre = int_w(0)
    x, t = _upd_ext_pre(x, pos_p, part, part, wu1[0:64], wu1[64:128],
                        p['ex1_u1']['b'][None, :], p['ex1_u2']['w'],
                        p['ex1_u2']['b'][None, :], w1c, w2c, bpre)

    for i in range(_NPASS):
        pa, pb = msg_round_split(t, p['int'][i]['m2'])
        lp = p['int'][i]
        wu1 = lp['u1']['w']
        if i + 1 < _NPASS:
            w1c, w2c, bpre = int_w(i + 1)
        else:
            w1c, w2c, bpre = zero66, wxc2, bpre2
        x, t = _upd_int_pre(x, u_p, pos_p, pa, pb, wu1[0:64], wu1[64:128],
                            wu1[128:176], lp['u1']['b'][None, :],
                            lp['u2']['w'], lp['u2']['b'][None, :],
                            w1c, w2c, bpre)

    t_e = _pre_ext_e(ef18, wec2)
    part = msg_round(gather2, scatter2, t_e, t, src2, dst2, p['ex2_m2'],
                     _E2_PAD, 4096)
    wu1 = p['ex2_u1']['w']
    out = _upd_ext_out(x, part, part, wu1[0:64], wu1[64:128],
                       p['ex2_u1']['b'][None, :], p['ex2_u2']['w'],
                       p['ex2_u2']['b'][None, :],
                       p['out1']['w'], p['out1']['b'][None, :],
                       p['out2']['w'], p['out2']['b'][None, :])
    return out[0:_N].reshape(_B, _NM, _NOUT)
